# Initial kernel scaffold; baseline (speedup 1.0000x reference)
import jax
import jax.numpy as jnp
from jax.experimental import pallas as pl


def kernel(gt, pred, idx12):
    raise NotImplementedError("write your pallas kernel here")


# speedup vs baseline: 1.0008x; 1.0008x over previous
import jax
import jax.numpy as jnp
from jax.experimental import pallas as pl

_K = 16
_EPS = 1e-8


def _knn_idx(points, k):
    sq = jnp.sum(points ** 2, axis=-1)
    inner = jnp.einsum('bnd,bmd->bnm', points, points)
    dist = sq[:, :, None] - 2.0 * inner + sq[:, None, :]
    _, idx = jax.lax.top_k(-dist, k)
    return idx


def _batch_normals(points, k):
    idx = _knn_idx(points, k)
    grouped = jax.vmap(lambda p, i: p[i])(points, idx)
    centered = grouped - jnp.mean(grouped, axis=2, keepdims=True)
    cov = jnp.einsum('bnki,bnkj->bnij', centered, centered) / k
    eigval, eigvec = jnp.linalg.eigh(cov)
    return eigvec[..., :, 0]


def _loss_kernel(a_ref, b_ref, out_ref):
    a = a_ref[...]
    b = b_ref[...]
    num = jnp.sum(a * b, axis=-1)
    na = jnp.maximum(jnp.sqrt(jnp.sum(a * a, axis=-1)), _EPS)
    nb = jnp.maximum(jnp.sqrt(jnp.sum(b * b, axis=-1)), _EPS)
    loss = 1.0 - num / (na * nb)
    out_ref[...] = jnp.sum(loss).reshape(1, 1) * 0.25


def kernel(gt, pred, idx12):
    gt_normals = _batch_normals(gt, _K)
    pred_g = jnp.take_along_axis(pred, idx12[:, :, None].astype(jnp.int32), axis=1)
    pred_normals = _batch_normals(pred_g, _K)
    a = pred_normals.reshape(-1, 3)
    b = gt_normals.reshape(-1, 3)
    out = pl.pallas_call(
        _loss_kernel,
        out_shape=jax.ShapeDtypeStruct((1, 1), jnp.float32),
    )(a, b)
    return out[0, 0]


# speedup vs baseline: 37.5653x; 37.5349x over previous
import functools

import jax
import jax.numpy as jnp
from jax.experimental import pallas as pl

_K = 16
_EPS = 1e-8
_N = 2048
_R = 256
_SWEEPS = 6


def _jacobi3_min_eigvec(a00, a11, a22, a01, a02, a12):
    A = [[a00, a01, a02], [a01, a11, a12], [a02, a12, a22]]
    one = jnp.ones_like(a00)
    zero = jnp.zeros_like(a00)
    V = [[one, zero, zero], [zero, one, zero], [zero, zero, one]]
    for _ in range(_SWEEPS):
        for (p, q) in ((0, 2), (1, 2), (0, 1)):
            app, aqq, apq = A[p][p], A[q][q], A[p][q]
            nz = apq != 0.0
            tau = (aqq - app) / (2.0 * apq)
            sgn = jnp.where(tau >= 0.0, 1.0, -1.0)
            t = jnp.where(nz, sgn / (jnp.abs(tau) + jnp.sqrt(1.0 + tau * tau)), 0.0)
            c = jax.lax.rsqrt(1.0 + t * t)
            s = t * c
            r = 3 - p - q
            arp, arq = A[r][p], A[r][q]
            new_app = c * (c * app - s * apq) - s * (c * apq - s * aqq)
            new_aqq = s * (s * app + c * apq) + c * (s * apq + c * aqq)
            new_apq = c * s * (app - aqq) + (c * c - s * s) * apq
            new_arp = c * arp - s * arq
            new_arq = s * arp + c * arq
            A[p][p] = new_app
            A[q][q] = new_aqq
            A[p][q] = new_apq
            A[q][p] = new_apq
            A[r][p] = new_arp
            A[p][r] = new_arp
            A[r][q] = new_arq
            A[q][r] = new_arq
            for i in range(3):
                vip, viq = V[i][p], V[i][q]
                V[i][p] = c * vip - s * viq
                V[i][q] = s * vip + c * viq
    w0, w1, w2 = A[0][0], A[1][1], A[2][2]
    sel0 = (w0 <= w1) & (w0 <= w2)
    sel1 = jnp.logical_not(sel0) & (w1 <= w2)
    nx = jnp.where(sel0, V[0][0], jnp.where(sel1, V[0][1], V[0][2]))
    ny = jnp.where(sel0, V[1][0], jnp.where(sel1, V[1][1], V[1][2]))
    nz_ = jnp.where(sel0, V[2][0], jnp.where(sel1, V[2][1], V[2][2]))
    return nx, ny, nz_


def _normals_for_cloud(xt, rt):
    x, y, z = xt[0:1, :], xt[1:2, :], xt[2:3, :]
    sq_all = (x * x + y * y) + z * z
    sq_all_col = sq_all.reshape(_N, 1)
    rx, ry, rz = rt[0:1, :], rt[1:2, :], rt[2:3, :]
    sq_rows = (rx * rx + ry * ry) + rz * rz
    inner = jax.lax.dot_general(
        xt, rt, (((0,), (0,)), ((), ())),
        preferred_element_type=jnp.float32)
    dist = (sq_rows - 2.0 * inner) + sq_all_col

    iota = jax.lax.broadcasted_iota(jnp.int32, (_N, _R), 0)
    mask = jnp.zeros((_N, _R), dtype=jnp.bool_)
    big = jnp.int32(_N)
    for _ in range(_K):
        m = jnp.min(dist, axis=0, keepdims=True)
        eq = dist == m
        jmin = jnp.min(jnp.where(eq, iota, big), axis=0, keepdims=True)
        sel = iota == jmin
        mask = mask | sel
        dist = jnp.where(sel, jnp.float32(jnp.inf), dist)

    maskf = mask.astype(jnp.float32)
    mom = jax.lax.dot_general(
        xt, maskf, (((1,), (0,)), ((), ())),
        preferred_element_type=jnp.float32)
    inv_k = jnp.float32(1.0 / _K)
    mx = mom[0:1, :] * inv_k
    my = mom[1:2, :] * inv_k
    mz = mom[2:3, :] * inv_k
    xc = x.reshape(_N, 1) - mx
    yc = y.reshape(_N, 1) - my
    zc = z.reshape(_N, 1) - mz
    xm = maskf * xc
    ym = maskf * yc
    zm = maskf * zc
    ones_row = jnp.ones((1, _N), dtype=jnp.float32)

    def _colsum(v):
        return jax.lax.dot_general(
            ones_row, v, (((1,), (0,)), ((), ())),
            preferred_element_type=jnp.float32)

    cxx = _colsum(xm * xc) * inv_k
    cyy = _colsum(ym * yc) * inv_k
    czz = _colsum(zm * zc) * inv_k
    cxy = _colsum(xm * yc) * inv_k
    cxz = _colsum(xm * zc) * inv_k
    cyz = _colsum(ym * zc) * inv_k
    return _jacobi3_min_eigvec(cxx, cyy, czz, cxy, cxz, cyz)


def _fused_kernel(gt_t_ref, gt_r_ref, pr_t_ref, pr_r_ref, out_ref):
    gx, gy, gz = _normals_for_cloud(gt_t_ref[0], gt_r_ref[0])
    px, py, pz = _normals_for_cloud(pr_t_ref[0], pr_r_ref[0])
    num = gx * px + gy * py + gz * pz
    na = jnp.maximum(jnp.sqrt(gx * gx + gy * gy + gz * gz), _EPS)
    nb = jnp.maximum(jnp.sqrt(px * px + py * py + pz * pz), _EPS)
    loss = 1.0 - num / (na * nb)
    s = jnp.sum(loss)
    out_ref[...] = jnp.broadcast_to(s.reshape(1, 1, 1, 1), (1, 1, 1, 128))


@jax.jit
def kernel(gt, pred, idx12):
    B, N, _ = gt.shape
    nb = N // _R
    pred_g = jnp.take_along_axis(pred, idx12[:, :, None].astype(jnp.int32), axis=1)
    gt_t = jnp.swapaxes(gt, 1, 2)
    pr_t = jnp.swapaxes(pred_g, 1, 2)

    grid = (B, nb)
    full_spec = pl.BlockSpec((1, 3, N), lambda b, j: (b, 0, 0))
    rows_spec = pl.BlockSpec((1, 3, _R), lambda b, j: (b, 0, j))
    out_spec = pl.BlockSpec((1, 1, 1, 128), lambda b, j: (b, j, 0, 0))
    partial = pl.pallas_call(
        _fused_kernel,
        grid=grid,
        in_specs=[full_spec, rows_spec, full_spec, rows_spec],
        out_specs=out_spec,
        out_shape=jax.ShapeDtypeStruct((B, nb, 1, 128), jnp.float32),
    )(gt_t, gt_t, pr_t, pr_t)
    return jnp.sum(partial[:, :, 0, 0]) / B


# speedup vs baseline: 81.4181x; 2.1674x over previous
import functools

import jax
import jax.numpy as jnp
from jax.experimental import pallas as pl
from jax.experimental.pallas import tpu as pltpu

_K = 16
_EPS = 1e-8
_N = 2048
_R = 256
_SWEEPS = 6


def _jacobi3_min_eigvec(a00, a11, a22, a01, a02, a12):
    A = [[a00, a01, a02], [a01, a11, a12], [a02, a12, a22]]
    one = jnp.ones_like(a00)
    zero = jnp.zeros_like(a00)
    V = [[one, zero, zero], [zero, one, zero], [zero, zero, one]]
    for _ in range(_SWEEPS):
        for (p, q) in ((0, 2), (1, 2), (0, 1)):
            app, aqq, apq = A[p][p], A[q][q], A[p][q]
            nz = apq != 0.0
            tau = (aqq - app) / (2.0 * apq)
            sgn = jnp.where(tau >= 0.0, 1.0, -1.0)
            t = jnp.where(nz, sgn / (jnp.abs(tau) + jnp.sqrt(1.0 + tau * tau)), 0.0)
            c = jax.lax.rsqrt(1.0 + t * t)
            s = t * c
            r = 3 - p - q
            arp, arq = A[r][p], A[r][q]
            new_app = c * (c * app - s * apq) - s * (c * apq - s * aqq)
            new_aqq = s * (s * app + c * apq) + c * (s * apq + c * aqq)
            new_apq = c * s * (app - aqq) + (c * c - s * s) * apq
            new_arp = c * arp - s * arq
            new_arq = s * arp + c * arq
            A[p][p] = new_app
            A[q][q] = new_aqq
            A[p][q] = new_apq
            A[q][p] = new_apq
            A[r][p] = new_arp
            A[p][r] = new_arp
            A[r][q] = new_arq
            A[q][r] = new_arq
            for i in range(3):
                vip, viq = V[i][p], V[i][q]
                V[i][p] = c * vip - s * viq
                V[i][q] = s * vip + c * viq
    w0, w1, w2 = A[0][0], A[1][1], A[2][2]
    sel0 = (w0 <= w1) & (w0 <= w2)
    sel1 = jnp.logical_not(sel0) & (w1 <= w2)
    nx = jnp.where(sel0, V[0][0], jnp.where(sel1, V[0][1], V[0][2]))
    ny = jnp.where(sel0, V[1][0], jnp.where(sel1, V[1][1], V[1][2]))
    nz_ = jnp.where(sel0, V[2][0], jnp.where(sel1, V[2][1], V[2][2]))
    return nx, ny, nz_


def _normals_for_cloud(xt, rt):
    x, y, z = xt[0:1, :], xt[1:2, :], xt[2:3, :]
    sq_all = (x * x + y * y) + z * z
    sq_all_col = sq_all.reshape(_N, 1)
    rx, ry, rz = rt[0:1, :], rt[1:2, :], rt[2:3, :]
    sq_rows = (rx * rx + ry * ry) + rz * rz
    inner = jax.lax.dot_general(
        xt, rt, (((0,), (0,)), ((), ())),
        preferred_element_type=jnp.float32)
    dist = (sq_rows - 2.0 * inner) + sq_all_col

    iota = jax.lax.broadcasted_iota(jnp.int32, (_N, _R), 0)
    inf = jnp.float32(jnp.inf)
    for _ in range(_K):
        jmin = jnp.argmin(dist, axis=0).reshape(1, _R)
        sel = iota == jmin
        dist = jnp.where(sel, inf, dist)

    maskf = (dist == inf).astype(jnp.float32)
    mom = jax.lax.dot_general(
        xt, maskf, (((1,), (0,)), ((), ())),
        preferred_element_type=jnp.float32)
    inv_k = jnp.float32(1.0 / _K)
    mx = mom[0:1, :] * inv_k
    my = mom[1:2, :] * inv_k
    mz = mom[2:3, :] * inv_k
    xc = x.reshape(_N, 1) - mx
    yc = y.reshape(_N, 1) - my
    zc = z.reshape(_N, 1) - mz
    xm = maskf * xc
    ym = maskf * yc
    zm = maskf * zc
    ones_row = jnp.ones((1, _N), dtype=jnp.float32)

    def _colsum(v):
        return jax.lax.dot_general(
            ones_row, v, (((1,), (0,)), ((), ())),
            preferred_element_type=jnp.float32)

    cxx = _colsum(xm * xc) * inv_k
    cyy = _colsum(ym * yc) * inv_k
    czz = _colsum(zm * zc) * inv_k
    cxy = _colsum(xm * yc) * inv_k
    cxz = _colsum(xm * zc) * inv_k
    cyz = _colsum(ym * zc) * inv_k
    return _jacobi3_min_eigvec(cxx, cyy, czz, cxy, cxz, cyz)


def _fused_kernel(gt_t_ref, gt_r_ref, pr_t_ref, pr_r_ref, out_ref):
    gx, gy, gz = _normals_for_cloud(gt_t_ref[0], gt_r_ref[0])
    px, py, pz = _normals_for_cloud(pr_t_ref[0], pr_r_ref[0])
    num = gx * px + gy * py + gz * pz
    na = jnp.maximum(jnp.sqrt(gx * gx + gy * gy + gz * gz), _EPS)
    nb = jnp.maximum(jnp.sqrt(px * px + py * py + pz * pz), _EPS)
    loss = 1.0 - num / (na * nb)
    s = jnp.sum(loss)
    out_ref[...] = jnp.broadcast_to(s.reshape(1, 1, 1, 1), (1, 1, 1, 128))


@jax.jit
def kernel(gt, pred, idx12):
    B, N, _ = gt.shape
    nb = N // _R
    pred_g = jnp.take_along_axis(pred, idx12[:, :, None].astype(jnp.int32), axis=1)
    gt_t = jnp.swapaxes(gt, 1, 2)
    pr_t = jnp.swapaxes(pred_g, 1, 2)

    grid = (B, nb)
    full_spec = pl.BlockSpec((1, 3, N), lambda b, j: (b, 0, 0))
    rows_spec = pl.BlockSpec((1, 3, _R), lambda b, j: (b, 0, j))
    out_spec = pl.BlockSpec((1, 1, 1, 128), lambda b, j: (b, j, 0, 0))
    partial = pl.pallas_call(
        _fused_kernel,
        grid=grid,
        in_specs=[full_spec, rows_spec, full_spec, rows_spec],
        out_specs=out_spec,
        out_shape=jax.ShapeDtypeStruct((B, nb, 1, 128), jnp.float32),
        compiler_params=pltpu.CompilerParams(
            dimension_semantics=("parallel", "parallel")),
    )(gt_t, gt_t, pr_t, pr_t)
    return jnp.sum(partial[:, :, 0, 0]) / B
